# baseline (speedup 1.0000x reference)
import functools

import jax
import jax.numpy as jnp
from jax import lax
from jax.experimental import pallas as pl
from jax.experimental.pallas import tpu as pltpu
from jax.experimental.pallas import tpu_sc as plsc

B = 4096
F = 26
V = 100000
D = 32
BN_EPS = 1e-3
BN_SCALE = 1.0 / (1.0 + BN_EPS) ** 0.5

NC = 2
NS = 16
NW = NC * NS
BF = B * F
PER_W = BF // NW
CHUNK = 128
NCHUNK = PER_W // CHUNK
GROUPS = CHUNK // 16


def _sc_mesh():
    return plsc.VectorSubcoreMesh(core_axis_name="c", subcore_axis_name="s")


@functools.partial(
    pl.kernel,
    mesh=_sc_mesh(),
    compiler_params=pltpu.CompilerParams(
        use_tc_tiling_on_sc=False, needs_layout_passes=False
    ),
    out_type=jax.ShapeDtypeStruct((BF,), jnp.float32),
    scratch_types=[
        pltpu.VMEM((NCHUNK, CHUNK), jnp.int32),
        pltpu.VMEM((2 * CHUNK, D), jnp.float32),
        pltpu.VMEM((PER_W,), jnp.float32),
        pltpu.SemaphoreType.DMA,
        pltpu.SemaphoreType.DMA,
    ],
)
def _sc_gather_bi(table_hbm, idx_hbm, out_hbm, idx_v, rows_v, bi_v, sem0, sem1):
    cid = lax.axis_index("c")
    sid = lax.axis_index("s")
    wid = sid * NC + cid

    pltpu.sync_copy(idx_hbm.at[wid], idx_v)

    sems = (sem0, sem1)

    def rbuf(buf):
        return rows_v.at[pl.ds(buf * CHUNK, CHUNK)]

    def start(j, buf):
        pltpu.async_copy(table_hbm.at[idx_v.at[j]], rbuf(buf), sems[buf])

    def wait(j, buf):
        pltpu.make_async_copy(
            table_hbm.at[idx_v.at[j]], rbuf(buf), sems[buf]
        ).wait()

    lane = lax.iota(jnp.int32, 16)

    def compute(j, buf):
        def group(g, carry):
            row0 = g * 16
            ridx = buf * CHUNK + row0 + lane
            acc = jnp.zeros((16,), jnp.float32)
            acc2 = jnp.zeros((16,), jnp.float32)
            for d in range(D):
                col = jnp.full((16,), d, jnp.int32)
                v = plsc.load_gather(rows_v, [ridx, col])
                acc = acc + v
                acc2 = acc2 + v * v
            bi_v[pl.ds(j * CHUNK + row0, 16)] = 0.5 * (acc * acc - acc2)
            return carry
        lax.fori_loop(0, GROUPS, group, 0)

    start(0, 0)

    def pair(j, carry):
        start(j + 1, 1)
        wait(j, 0)
        compute(j, 0)

        @pl.when(j + 2 < NCHUNK)
        def _():
            start(j + 2, 0)

        wait(j + 1, 1)
        compute(j + 1, 1)
        return carry

    lax.fori_loop(0, NCHUNK // 2, lambda i, c: pair(i * 2, c), 0)

    pltpu.sync_copy(bi_v, out_hbm.at[pl.ds(wid * PER_W, PER_W)])


BLK = 1024


def _mlp_body(bi_ref, gamma_ref, beta_ref, w1_ref, b1_ref, w2_ref, b2_ref,
              w3_ref, b3_ref, wout_ref, out_ref):
    bn = bi_ref[...] * (gamma_ref[...] * BN_SCALE) + beta_ref[...]
    h = jnp.dot(bn, w1_ref[...], preferred_element_type=jnp.float32)
    h = jnp.maximum(h + b1_ref[...], 0.0)
    h = jnp.dot(h, w2_ref[...], preferred_element_type=jnp.float32)
    h = jnp.maximum(h + b2_ref[...], 0.0)
    h = jnp.dot(h, w3_ref[...], preferred_element_type=jnp.float32)
    h = jnp.maximum(h + b3_ref[...], 0.0)
    y = jnp.dot(h, wout_ref[...], preferred_element_type=jnp.float32)
    out_ref[...] = jax.nn.sigmoid(y)


def _rep(shape):
    return pl.BlockSpec(shape, lambda i: (0,) * len(shape))


def _mlp(bi, bn_gamma, bn_beta, W1, b1, W2, b2, W3, b3, Wout):
    h1, h2, h3 = W1.shape[1], W2.shape[1], W3.shape[1]
    return pl.pallas_call(
        _mlp_body,
        grid=(B // BLK,),
        in_specs=[
            pl.BlockSpec((BLK, F), lambda i: (i, 0)),
            _rep((1, F)),
            _rep((1, F)),
            _rep((F, h1)),
            _rep((1, h1)),
            _rep((h1, h2)),
            _rep((1, h2)),
            _rep((h2, h3)),
            _rep((1, h3)),
            _rep((h3, 1)),
        ],
        out_specs=pl.BlockSpec((BLK, 1), lambda i: (i, 0)),
        out_shape=jax.ShapeDtypeStruct((B, 1), jnp.float32),
    )(bi, bn_gamma.reshape(1, F), bn_beta.reshape(1, F), W1,
      b1.reshape(1, h1), W2, b2.reshape(1, h2), W3, b3.reshape(1, h3), Wout)


@jax.jit
def kernel(inputs, tables, bn_gamma, bn_beta, W1, b1, W2, b2, W3, b3, Wout):
    flat_table = tables.reshape(F * V, D)
    flat_idx = (inputs + (jnp.arange(F, dtype=jnp.int32) * V)[None, :])
    flat_idx = flat_idx.reshape(NW, NCHUNK, CHUNK)
    bi = _sc_gather_bi(flat_table, flat_idx).reshape(B, F)
    return _mlp(bi, bn_gamma, bn_beta, W1, b1, W2, b2, W3, b3, Wout)
